# baseline (device time: 93034 ns/iter reference)
import functools

import jax
import jax.numpy as jnp
from jax import lax
from jax.experimental import pallas as pl
from jax.experimental.pallas import tpu as pltpu

N_DEV = 16
N_WSLOT = 2
N_WCHUNK = 4
F32_END = 6
BF16_END = 10


def kernel(x, w_mat):
    m_loc, k = x.shape
    _, n = w_mat.shape
    n_loc = n // N_DEV
    m = m_loc * N_DEV
    rows = k // N_WCHUNK

    def body(x_ref, w_hbm, out_ref,
             w_vmem, blocks, recv_f32, send_bf16, recv_bf16,
             send_i8, recv_i8, amax_buf,
             w_sems, send_sems, recv_sems, amax_send_sems, amax_recv_sems):
        my_pos = lax.axis_index("i")

        barrier = pltpu.get_barrier_semaphore()
        for p in range(N_DEV):
            pl.semaphore_signal(
                barrier, inc=1,
                device_id=(p,), device_id_type=pl.DeviceIdType.MESH,
            )

        def w_copies(s):
            dest = (my_pos + s) % N_DEV
            return [
                pltpu.make_async_copy(
                    w_hbm.at[pl.ds(c * rows, rows), pl.ds(dest * n_loc, n_loc)],
                    w_vmem.at[s % N_WSLOT, pl.ds(c * rows, rows)],
                    w_sems.at[s % N_WSLOT, c],
                )
                for c in range(N_WCHUNK)
            ]

        for s in range(N_WSLOT):
            for cp in w_copies(s):
                cp.start()
        amax_parts = []
        f32_rdmas = []
        for s in range(N_DEV):
            for cp in w_copies(s):
                cp.wait()
            if s + N_WSLOT < N_DEV:
                for cp in w_copies(s + N_WSLOT):
                    cp.start()
            y = jnp.dot(x_ref[:, :], w_vmem[s % N_WSLOT],
                        preferred_element_type=jnp.float32)
            amax_parts.append(jnp.max(jnp.abs(y)))
            blocks[s] = y
            if s == 1:
                pl.semaphore_wait(barrier, N_DEV)
            if 1 <= s < BF16_END:
                dest = (my_pos + s) % N_DEV
                if s < F32_END:
                    src, dst = blocks.at[s], recv_f32.at[s]
                else:
                    send_bf16[s] = y.astype(jnp.bfloat16)
                    src, dst = send_bf16.at[s], recv_bf16.at[s]
                r = pltpu.make_async_remote_copy(
                    src_ref=src,
                    dst_ref=dst,
                    send_sem=send_sems.at[s],
                    recv_sem=recv_sems.at[s],
                    device_id=(dest,),
                    device_id_type=pl.DeviceIdType.MESH,
                )
                r.start()
                f32_rdmas.append(r)

        my_amax = functools.reduce(jnp.maximum, amax_parts)
        amax_buf[0] = jnp.full((8, 128), my_amax, jnp.float32)
        amax_rdmas = []
        for s in range(1, N_DEV):
            dest = (my_pos + s) % N_DEV
            r = pltpu.make_async_remote_copy(
                src_ref=amax_buf.at[0],
                dst_ref=amax_buf.at[s],
                send_sem=amax_send_sems.at[s],
                recv_sem=amax_recv_sems.at[s],
                device_id=(dest,),
                device_id_type=pl.DeviceIdType.MESH,
            )
            r.start()
            amax_rdmas.append(r)
        for r in amax_rdmas:
            r.wait_recv()
        global_amax = jnp.max(amax_buf[:, :, :])
        scale = global_amax / 127.0

        def quant(y):
            return jnp.clip(jnp.round(y / scale), -127.0, 127.0).astype(jnp.int8)

        i8_rdmas = []
        for s in range(BF16_END, N_DEV):
            send_i8[s] = quant(blocks[s])
            dest = (my_pos + s) % N_DEV
            rdma = pltpu.make_async_remote_copy(
                src_ref=send_i8.at[s],
                dst_ref=recv_i8.at[s],
                send_sem=send_sems.at[s],
                recv_sem=recv_sems.at[s],
                device_id=(dest,),
                device_id_type=pl.DeviceIdType.MESH,
            )
            rdma.start()
            i8_rdmas.append(rdma)

        out_ref[pl.ds(my_pos * m_loc, m_loc), :] = (
            quant(blocks[0]).astype(jnp.float32) * scale)
        for s in range(1, BF16_END):
            f32_rdmas[s - 1].wait_recv()
            src = (my_pos - s) % N_DEV
            y_in = (recv_f32[s] if s < F32_END
                    else recv_bf16[s].astype(jnp.float32))
            out_ref[pl.ds(src * m_loc, m_loc), :] = (
                quant(y_in).astype(jnp.float32) * scale)
        for s in range(BF16_END, N_DEV):
            i8_rdmas[s - BF16_END].wait_recv()
            src = (my_pos - s) % N_DEV
            out_ref[pl.ds(src * m_loc, m_loc), :] = (
                recv_i8[s].astype(jnp.float32) * scale)

        for r in f32_rdmas:
            r.wait_send()
        for r in i8_rdmas:
            r.wait_send()
        for r in amax_rdmas:
            r.wait_send()

    return pl.pallas_call(
        body,
        out_shape=jax.ShapeDtypeStruct((m, n_loc), jnp.float32),
        in_specs=[
            pl.BlockSpec(memory_space=pltpu.VMEM),
            pl.BlockSpec(memory_space=pl.ANY),
        ],
        out_specs=pl.BlockSpec(memory_space=pltpu.VMEM),
        scratch_shapes=[
            pltpu.VMEM((N_WSLOT, k, n_loc), jnp.float32),
            pltpu.VMEM((N_DEV, m_loc, n_loc), jnp.float32),
            pltpu.VMEM((F32_END, m_loc, n_loc), jnp.float32),
            pltpu.VMEM((BF16_END, m_loc, n_loc), jnp.bfloat16),
            pltpu.VMEM((BF16_END, m_loc, n_loc), jnp.bfloat16),
            pltpu.VMEM((N_DEV, m_loc, n_loc), jnp.int8),
            pltpu.VMEM((N_DEV, m_loc, n_loc), jnp.int8),
            pltpu.VMEM((N_DEV, 8, 128), jnp.float32),
            pltpu.SemaphoreType.DMA((N_WSLOT, N_WCHUNK)),
            pltpu.SemaphoreType.DMA((N_DEV,)),
            pltpu.SemaphoreType.DMA((N_DEV,)),
            pltpu.SemaphoreType.DMA((N_DEV,)),
            pltpu.SemaphoreType.DMA((N_DEV,)),
        ],
        compiler_params=pltpu.CompilerParams(
            collective_id=0,
            vmem_limit_bytes=100 * 1024 * 1024,
        ),
    )(x, w_mat)


# device time: 79606 ns/iter; 1.1687x vs baseline; 1.1687x over previous
import functools

import jax
import jax.numpy as jnp
from jax import lax
from jax.experimental import pallas as pl
from jax.experimental.pallas import tpu as pltpu

N_DEV = 16
N_WSLOT = 2
N_WCHUNK = 4
F32_END = 6
BF16_END = 6


def kernel(x, w_mat):
    m_loc, k = x.shape
    _, n = w_mat.shape
    n_loc = n // N_DEV
    m = m_loc * N_DEV
    rows = k // N_WCHUNK

    def body(x_ref, w_hbm, out_ref,
             w_vmem, blocks, recv_f32, send_bf16, recv_bf16,
             send_i8, recv_i8, amax_buf,
             w_sems, send_sems, recv_sems, amax_send_sems, amax_recv_sems):
        my_pos = lax.axis_index("i")

        barrier = pltpu.get_barrier_semaphore()
        for p in range(N_DEV):
            pl.semaphore_signal(
                barrier, inc=1,
                device_id=(p,), device_id_type=pl.DeviceIdType.MESH,
            )

        def w_copies(s):
            dest = (my_pos + s) % N_DEV
            return [
                pltpu.make_async_copy(
                    w_hbm.at[pl.ds(c * rows, rows), pl.ds(dest * n_loc, n_loc)],
                    w_vmem.at[s % N_WSLOT, pl.ds(c * rows, rows)],
                    w_sems.at[s % N_WSLOT, c],
                )
                for c in range(N_WCHUNK)
            ]

        for s in range(N_WSLOT):
            for cp in w_copies(s):
                cp.start()
        amax_parts = []
        f32_rdmas = []
        for s in range(N_DEV):
            for cp in w_copies(s):
                cp.wait()
            if s + N_WSLOT < N_DEV:
                for cp in w_copies(s + N_WSLOT):
                    cp.start()
            y = jnp.dot(x_ref[:, :], w_vmem[s % N_WSLOT],
                        preferred_element_type=jnp.float32)
            amax_parts.append(jnp.max(jnp.abs(y)))
            blocks[s] = y
            if s == 1:
                pl.semaphore_wait(barrier, N_DEV)
            if 1 <= s < BF16_END:
                dest = (my_pos + s) % N_DEV
                if s < F32_END:
                    src, dst = blocks.at[s], recv_f32.at[s]
                else:
                    send_bf16[s] = y.astype(jnp.bfloat16)
                    src, dst = send_bf16.at[s], recv_bf16.at[s]
                r = pltpu.make_async_remote_copy(
                    src_ref=src,
                    dst_ref=dst,
                    send_sem=send_sems.at[s],
                    recv_sem=recv_sems.at[s],
                    device_id=(dest,),
                    device_id_type=pl.DeviceIdType.MESH,
                )
                r.start()
                f32_rdmas.append(r)

        my_amax = functools.reduce(jnp.maximum, amax_parts)
        amax_buf[0] = jnp.full((8, 128), my_amax, jnp.float32)
        amax_rdmas = []
        for s in range(1, N_DEV):
            dest = (my_pos + s) % N_DEV
            r = pltpu.make_async_remote_copy(
                src_ref=amax_buf.at[0],
                dst_ref=amax_buf.at[s],
                send_sem=amax_send_sems.at[s],
                recv_sem=amax_recv_sems.at[s],
                device_id=(dest,),
                device_id_type=pl.DeviceIdType.MESH,
            )
            r.start()
            amax_rdmas.append(r)
        for r in amax_rdmas:
            r.wait_recv()
        global_amax = jnp.max(amax_buf[:, :, :])
        scale = global_amax / 127.0

        def quant(y):
            return jnp.clip(jnp.round(y / scale), -127.0, 127.0).astype(jnp.int8)

        i8_rdmas = []
        for s in range(BF16_END, N_DEV):
            send_i8[s] = quant(blocks[s])
            dest = (my_pos + s) % N_DEV
            rdma = pltpu.make_async_remote_copy(
                src_ref=send_i8.at[s],
                dst_ref=recv_i8.at[s],
                send_sem=send_sems.at[s],
                recv_sem=recv_sems.at[s],
                device_id=(dest,),
                device_id_type=pl.DeviceIdType.MESH,
            )
            rdma.start()
            i8_rdmas.append(rdma)

        out_ref[pl.ds(my_pos * m_loc, m_loc), :] = (
            quant(blocks[0]).astype(jnp.float32) * scale)
        for s in range(1, BF16_END):
            f32_rdmas[s - 1].wait_recv()
            src = (my_pos - s) % N_DEV
            y_in = (recv_f32[s] if s < F32_END
                    else recv_bf16[s].astype(jnp.float32))
            out_ref[pl.ds(src * m_loc, m_loc), :] = (
                quant(y_in).astype(jnp.float32) * scale)
        for s in range(BF16_END, N_DEV):
            i8_rdmas[s - BF16_END].wait_recv()
            src = (my_pos - s) % N_DEV
            out_ref[pl.ds(src * m_loc, m_loc), :] = (
                recv_i8[s].astype(jnp.float32) * scale)

        for r in f32_rdmas:
            r.wait_send()
        for r in i8_rdmas:
            r.wait_send()
        for r in amax_rdmas:
            r.wait_send()

    return pl.pallas_call(
        body,
        out_shape=jax.ShapeDtypeStruct((m, n_loc), jnp.float32),
        in_specs=[
            pl.BlockSpec(memory_space=pltpu.VMEM),
            pl.BlockSpec(memory_space=pl.ANY),
        ],
        out_specs=pl.BlockSpec(memory_space=pltpu.VMEM),
        scratch_shapes=[
            pltpu.VMEM((N_WSLOT, k, n_loc), jnp.float32),
            pltpu.VMEM((N_DEV, m_loc, n_loc), jnp.float32),
            pltpu.VMEM((F32_END, m_loc, n_loc), jnp.float32),
            pltpu.VMEM((BF16_END, m_loc, n_loc), jnp.bfloat16),
            pltpu.VMEM((BF16_END, m_loc, n_loc), jnp.bfloat16),
            pltpu.VMEM((N_DEV, m_loc, n_loc), jnp.int8),
            pltpu.VMEM((N_DEV, m_loc, n_loc), jnp.int8),
            pltpu.VMEM((N_DEV, 8, 128), jnp.float32),
            pltpu.SemaphoreType.DMA((N_WSLOT, N_WCHUNK)),
            pltpu.SemaphoreType.DMA((N_DEV,)),
            pltpu.SemaphoreType.DMA((N_DEV,)),
            pltpu.SemaphoreType.DMA((N_DEV,)),
            pltpu.SemaphoreType.DMA((N_DEV,)),
        ],
        compiler_params=pltpu.CompilerParams(
            collective_id=0,
            vmem_limit_bytes=100 * 1024 * 1024,
        ),
    )(x, w_mat)
